# baseline (device time: 84006 ns/iter reference)
import jax
import jax.numpy as jnp
from jax import lax
from jax.experimental import pallas as pl
from jax.experimental.pallas import tpu as pltpu

N_DEV = 32
M = 768
D = 768
CHUNK = M // N_DEV


def kernel(x, Wg, Wu, Wd):
    def body(x_ref, wg_ref, wu_ref, wd_ref, out_ref,
             partial_ref, comm_ref,
             send1_sems, recv1_sems, send2_sems, recv2_sems):
        me = lax.axis_index("i")

        barrier = pltpu.get_barrier_semaphore()
        for d in range(1, N_DEV):
            peer = lax.rem(me + d, N_DEV)
            pl.semaphore_signal(barrier, inc=1, device_id=(peer,),
                                device_id_type=pl.DeviceIdType.MESH)
        pl.semaphore_wait(barrier, N_DEV - 1)

        xb = x_ref[...].astype(jnp.bfloat16)
        gate = jnp.dot(xb, wg_ref[...].astype(jnp.bfloat16),
                       preferred_element_type=jnp.float32)
        up = jnp.dot(xb, wu_ref[...].astype(jnp.bfloat16),
                     preferred_element_type=jnp.float32)
        act = (gate * (up * jax.nn.sigmoid(up))).astype(jnp.bfloat16)
        partial = jnp.dot(act, wd_ref[...].astype(jnp.bfloat16),
                          preferred_element_type=jnp.float32)
        for c in range(N_DEV):
            partial_ref[c] = partial[c * CHUNK:(c + 1) * CHUNK, :]

        sends1 = []
        for d in range(1, N_DEV):
            peer = lax.rem(me + d, N_DEV)
            rdma = pltpu.make_async_remote_copy(
                src_ref=partial_ref.at[peer],
                dst_ref=comm_ref.at[me],
                send_sem=send1_sems.at[peer],
                recv_sem=recv1_sems.at[me],
                device_id=(peer,),
                device_id_type=pl.DeviceIdType.MESH,
            )
            rdma.start()
            sends1.append(rdma)

        comm_ref[pl.ds(me, 1)] = partial_ref[pl.ds(me, 1)]

        for d in range(1, N_DEV):
            src = lax.rem(me + d, N_DEV)
            pltpu.make_async_remote_copy(
                src_ref=partial_ref.at[src],
                dst_ref=comm_ref.at[src],
                send_sem=send1_sems.at[src],
                recv_sem=recv1_sems.at[src],
                device_id=(src,),
                device_id_type=pl.DeviceIdType.MESH,
            ).wait_recv()
        for rdma in sends1:
            rdma.wait_send()

        reduced = jnp.sum(comm_ref[...], axis=0)
        out_ref[pl.ds(me * CHUNK, CHUNK), :] = reduced

        sends2 = []
        for d in range(1, N_DEV):
            peer = lax.rem(me + d, N_DEV)
            rdma = pltpu.make_async_remote_copy(
                src_ref=out_ref.at[pl.ds(me * CHUNK, CHUNK), :],
                dst_ref=out_ref.at[pl.ds(me * CHUNK, CHUNK), :],
                send_sem=send2_sems.at[peer],
                recv_sem=recv2_sems.at[me],
                device_id=(peer,),
                device_id_type=pl.DeviceIdType.MESH,
            )
            rdma.start()
            sends2.append(rdma)
        for d in range(1, N_DEV):
            src = lax.rem(me + d, N_DEV)
            pltpu.make_async_remote_copy(
                src_ref=out_ref.at[pl.ds(src * CHUNK, CHUNK), :],
                dst_ref=out_ref.at[pl.ds(src * CHUNK, CHUNK), :],
                send_sem=send2_sems.at[src],
                recv_sem=recv2_sems.at[src],
                device_id=(src,),
                device_id_type=pl.DeviceIdType.MESH,
            ).wait_recv()
        for rdma in sends2:
            rdma.wait_send()

    return pl.pallas_call(
        body,
        out_shape=jax.ShapeDtypeStruct((M, D), jnp.float32),
        in_specs=[pl.BlockSpec(memory_space=pltpu.VMEM)] * 4,
        out_specs=pl.BlockSpec(memory_space=pltpu.VMEM),
        scratch_shapes=[
            pltpu.VMEM((N_DEV, CHUNK, D), jnp.float32),
            pltpu.VMEM((N_DEV, CHUNK, D), jnp.float32),
            pltpu.SemaphoreType.DMA((N_DEV,)),
            pltpu.SemaphoreType.DMA((N_DEV,)),
            pltpu.SemaphoreType.DMA((N_DEV,)),
            pltpu.SemaphoreType.DMA((N_DEV,)),
        ],
        compiler_params=pltpu.CompilerParams(collective_id=0),
    )(x, Wg, Wu, Wd)


# device time: 55138 ns/iter; 1.5236x vs baseline; 1.5236x over previous
import jax
import jax.numpy as jnp
from jax import lax
from jax.experimental import pallas as pl
from jax.experimental.pallas import tpu as pltpu

N_DEV = 32
M = 768
D = 768
CHUNK = M // N_DEV


def kernel(x, Wg, Wu, Wd):
    def body(x_ref, wg_ref, wu_ref, wd_ref, out_ref,
             partial_ref, comm_ref, gather_ref,
             send1_sems, recv1_sems, send2_sems, recv2_sems):
        me = lax.axis_index("i")

        barrier = pltpu.get_barrier_semaphore()
        for d in range(1, N_DEV):
            peer = lax.rem(me + d, N_DEV)
            pl.semaphore_signal(barrier, inc=1, device_id=(peer,),
                                device_id_type=pl.DeviceIdType.MESH)
        pl.semaphore_wait(barrier, N_DEV - 1)

        xb = x_ref[...].astype(jnp.bfloat16)
        gate = jnp.dot(xb, wg_ref[...].astype(jnp.bfloat16),
                       preferred_element_type=jnp.float32)
        up = jnp.dot(xb, wu_ref[...].astype(jnp.bfloat16),
                     preferred_element_type=jnp.float32)
        act = (gate * (up * jax.nn.sigmoid(up))).astype(jnp.bfloat16)
        partial = jnp.dot(act, wd_ref[...].astype(jnp.bfloat16),
                          preferred_element_type=jnp.float32
                          ).astype(jnp.bfloat16)
        for c in range(N_DEV):
            partial_ref[c] = partial[c * CHUNK:(c + 1) * CHUNK, :]

        sends1 = []
        for d in range(1, N_DEV):
            peer = lax.rem(me + d, N_DEV)
            rdma = pltpu.make_async_remote_copy(
                src_ref=partial_ref.at[peer],
                dst_ref=comm_ref.at[me],
                send_sem=send1_sems.at[peer],
                recv_sem=recv1_sems.at[me],
                device_id=(peer,),
                device_id_type=pl.DeviceIdType.MESH,
            )
            rdma.start()
            sends1.append(rdma)

        comm_ref[pl.ds(me, 1)] = partial_ref[pl.ds(me, 1)]

        for d in range(1, N_DEV):
            src = lax.rem(me + d, N_DEV)
            pltpu.make_async_remote_copy(
                src_ref=partial_ref.at[src],
                dst_ref=comm_ref.at[src],
                send_sem=send1_sems.at[src],
                recv_sem=recv1_sems.at[src],
                device_id=(src,),
                device_id_type=pl.DeviceIdType.MESH,
            ).wait_recv()
        for rdma in sends1:
            rdma.wait_send()

        reduced = jnp.sum(comm_ref[...].astype(jnp.float32), axis=0)
        gather_ref[pl.ds(me, 1)] = reduced.astype(jnp.bfloat16).reshape(
            1, CHUNK, D)

        sends2 = []
        for d in range(1, N_DEV):
            peer = lax.rem(me + d, N_DEV)
            rdma = pltpu.make_async_remote_copy(
                src_ref=gather_ref.at[me],
                dst_ref=gather_ref.at[me],
                send_sem=send2_sems.at[peer],
                recv_sem=recv2_sems.at[me],
                device_id=(peer,),
                device_id_type=pl.DeviceIdType.MESH,
            )
            rdma.start()
            sends2.append(rdma)
        for d in range(1, N_DEV):
            src = lax.rem(me + d, N_DEV)
            pltpu.make_async_remote_copy(
                src_ref=gather_ref.at[src],
                dst_ref=gather_ref.at[src],
                send_sem=send2_sems.at[src],
                recv_sem=recv2_sems.at[src],
                device_id=(src,),
                device_id_type=pl.DeviceIdType.MESH,
            ).wait_recv()
        for c in range(N_DEV):
            out_ref[c * CHUNK:(c + 1) * CHUNK, :] = (
                gather_ref[c].astype(jnp.float32))
        for rdma in sends2:
            rdma.wait_send()

    return pl.pallas_call(
        body,
        out_shape=jax.ShapeDtypeStruct((M, D), jnp.float32),
        in_specs=[pl.BlockSpec(memory_space=pltpu.VMEM)] * 4,
        out_specs=pl.BlockSpec(memory_space=pltpu.VMEM),
        scratch_shapes=[
            pltpu.VMEM((N_DEV, CHUNK, D), jnp.bfloat16),
            pltpu.VMEM((N_DEV, CHUNK, D), jnp.bfloat16),
            pltpu.VMEM((N_DEV, CHUNK, D), jnp.bfloat16),
            pltpu.SemaphoreType.DMA((N_DEV,)),
            pltpu.SemaphoreType.DMA((N_DEV,)),
            pltpu.SemaphoreType.DMA((N_DEV,)),
            pltpu.SemaphoreType.DMA((N_DEV,)),
        ],
        compiler_params=pltpu.CompilerParams(collective_id=0),
    )(x, Wg, Wu, Wd)
